# Initial kernel scaffold; baseline (speedup 1.0000x reference)
import jax
import jax.numpy as jnp
from jax.experimental import pallas as pl


def kernel(boxes, scores, labels, feat_map, Wp):
    raise NotImplementedError("write your pallas kernel here")


# speedup vs baseline: 100.6858x; 100.6858x over previous
import functools

import jax
import jax.numpy as jnp
from jax import lax
from jax.experimental import pallas as pl
from jax.experimental.pallas import tpu as pltpu
from jax.experimental.pallas import tpu_sc as plsc

N = 5000
NPAD = 5120
K = 100
KPAD = 112
D = 256
NI = 117
GRID = 50
IMG = 800.0
HUMAN_IDX = 1
NMS_THRESH = 0.5
L = 16
NSLC = KPAD // L


def _sread(ref, i):
    return ref[pl.ds(i, L)][0]


def _swrite(ref, i, val):
    idx = jnp.full((L,), i, jnp.int32)
    lane = lax.broadcasted_iota(jnp.int32, (L,), 0)
    plsc.store_scatter(ref, [idx], jnp.full((L,), val), mask=lane == 0)


def _sc_nms_gather(bx1, by1, bx2, by2, area, sc, cell, lb, feat):
    mesh = plsc.VectorSubcoreMesh(
        core_axis_name="c", subcore_axis_name="s", num_cores=2, num_subcores=16
    )

    @functools.partial(
        pl.kernel,
        out_type=[
            jax.ShapeDtypeStruct((KPAD, D), jnp.float32),
            jax.ShapeDtypeStruct((KPAD,), jnp.float32),
            jax.ShapeDtypeStruct((KPAD,), jnp.float32),
        ],
        mesh=mesh,
        scratch_types=[
            pltpu.VMEM((NPAD,), jnp.float32),
            pltpu.VMEM((NPAD,), jnp.float32),
            pltpu.VMEM((NPAD,), jnp.float32),
            pltpu.VMEM((NPAD,), jnp.float32),
            pltpu.VMEM((NPAD,), jnp.float32),
            pltpu.VMEM((NPAD,), jnp.float32),
            pltpu.VMEM((NPAD,), jnp.int32),
            pltpu.VMEM((NPAD,), jnp.int32),
            pltpu.VMEM((KPAD,), jnp.float32),
            pltpu.VMEM((KPAD,), jnp.float32),
            pltpu.VMEM((KPAD,), jnp.float32),
            pltpu.VMEM((KPAD,), jnp.float32),
            pltpu.VMEM((KPAD,), jnp.float32),
            pltpu.VMEM((KPAD,), jnp.float32),
            pltpu.VMEM((KPAD,), jnp.float32),
            pltpu.VMEM((KPAD,), jnp.int32),
            pltpu.VMEM((KPAD, D), jnp.float32),
            pltpu.SemaphoreType.DMA,
        ],
        compiler_params=pltpu.CompilerParams(needs_layout_passes=False),
    )
    def k(h_x1, h_y1, h_x2, h_y2, h_a, h_sc, h_cell, h_lb, h_feat,
          o_f, o_ph, o_po,
          v_x1, v_y1, v_x2, v_y2, v_a, v_sc, v_cell_all, v_lb,
          g_x1, g_y1, g_x2, g_y2, g_a, v_ph, v_po, v_cell, v_frows, sem):
        wid = lax.axis_index("s") * 2 + lax.axis_index("c")

        @pl.when(wid == 0)
        def _():
            pltpu.sync_copy(h_x1, v_x1)
            pltpu.sync_copy(h_y1, v_y1)
            pltpu.sync_copy(h_x2, v_x2)
            pltpu.sync_copy(h_y2, v_y2)
            pltpu.sync_copy(h_a, v_a)
            pltpu.sync_copy(h_sc, v_sc)
            pltpu.sync_copy(h_cell, v_cell_all)
            pltpu.sync_copy(h_lb, v_lb)
            zf = jnp.zeros((L,), jnp.float32)
            zi = jnp.zeros((L,), jnp.int32)
            for g in range(NSLC):
                sl = pl.ds(g * L, L)
                g_x1[sl] = zf
                g_y1[sl] = zf
                g_x2[sl] = zf
                g_y2[sl] = zf
                g_a[sl] = zf
                v_ph[sl] = zf
                v_po[sl] = zf
                v_cell[sl] = zi

            def cond(st):
                i, cnt = st
                return jnp.logical_and(i < N, cnt < K)

            def body(st):
                i, cnt = st
                x1 = _sread(v_x1, i)
                y1 = _sread(v_y1, i)
                x2 = _sread(v_x2, i)
                y2 = _sread(v_y2, i)
                a = _sread(v_a, i)
                m = jnp.zeros((L,), jnp.float32)
                for g in range(NSLC):
                    sl = pl.ds(g * L, L)
                    ix = jnp.maximum(
                        jnp.minimum(x2, g_x2[sl]) - jnp.maximum(x1, g_x1[sl]), 0.0)
                    iy = jnp.maximum(
                        jnp.minimum(y2, g_y2[sl]) - jnp.maximum(y1, g_y1[sl]), 0.0)
                    inter = ix * iy
                    denom = a + g_a[sl] - inter + 1e-8
                    m = jnp.maximum(m, inter - NMS_THRESH * denom)
                supp = jnp.max(m, axis=0) > 0.0

                @pl.when(jnp.logical_not(supp))
                def _acc():
                    _swrite(g_x1, cnt, x1)
                    _swrite(g_y1, cnt, y1)
                    _swrite(g_x2, cnt, x2)
                    _swrite(g_y2, cnt, y2)
                    _swrite(g_a, cnt, a)
                    s = _sread(v_sc, i)
                    _swrite(v_ph, cnt, jnp.where(_sread(v_lb, i) == HUMAN_IDX, s, 0.0))
                    _swrite(v_po, cnt, s)
                    _swrite(v_cell, cnt, _sread(v_cell_all, i))

                return (i + 1, jnp.where(supp, cnt, cnt + 1))

            lax.while_loop(cond, body, (jnp.int32(0), jnp.int32(0)))

            pltpu.async_copy(h_feat.at[v_cell], v_frows, sem).wait()
            pltpu.sync_copy(v_frows, o_f)
            pltpu.sync_copy(v_ph, o_ph)
            pltpu.sync_copy(v_po, o_po)

    return k(bx1, by1, bx2, by2, area, sc, cell, lb, feat)


def _tc_pair(f, Wp, ph_col, po_row):
    def body(f_ref, wp_ref, ph_ref, po_ref, out_ref):
        fv = f_ref[...]
        A = jnp.dot(fv, wp_ref[0:D, :], preferred_element_type=jnp.float32)
        B = jnp.dot(fv, wp_ref[D:2 * D, :], preferred_element_type=jnp.float32)
        logits = A[0:K][:, None, :] + B[0:K][None, :, :]
        prior = ph_ref[0:K, :] * po_ref[:, 0:K]
        out_ref[...] = (1.0 / (1.0 + jnp.exp(-logits))) * prior[:, :, None]

    return pl.pallas_call(
        body,
        out_shape=jax.ShapeDtypeStruct((K, K, NI), jnp.float32),
    )(f, Wp, ph_col, po_row)


def kernel(boxes, scores, labels, feat_map, Wp):
    order = jnp.argsort(-lax.stop_gradient(scores))
    b = boxes[order]
    sc = scores[order]
    lb = labels[order]
    off = lb.astype(b.dtype) * (IMG + 2.0)
    bb = b + off[:, None]
    area = (bb[:, 2] - bb[:, 0]) * (bb[:, 3] - bb[:, 1])
    cx = lax.stop_gradient((b[:, 0] + b[:, 2]) * 0.5)
    cy = lax.stop_gradient((b[:, 1] + b[:, 3]) * 0.5)
    gx = jnp.clip((cx / IMG * GRID).astype(jnp.int32), 0, GRID - 1)
    gy = jnp.clip((cy / IMG * GRID).astype(jnp.int32), 0, GRID - 1)
    cell = gy * GRID + gx

    def pad(v):
        return jnp.pad(v, (0, NPAD - N))

    f, ph, po = _sc_nms_gather(
        pad(bb[:, 0]), pad(bb[:, 1]), pad(bb[:, 2]), pad(bb[:, 3]),
        pad(area), pad(sc), pad(cell), pad(lb), feat_map)
    out3 = _tc_pair(f, Wp, ph.reshape(KPAD, 1), po.reshape(1, KPAD))
    return out3.reshape(K * K, NI)


# speedup vs baseline: 183.3106x; 1.8206x over previous
import functools

import jax
import jax.numpy as jnp
from jax import lax
from jax.experimental import pallas as pl
from jax.experimental.pallas import tpu as pltpu
from jax.experimental.pallas import tpu_sc as plsc

N = 5000
NPAD = 5120
K = 100
KPAD = 112
D = 256
NI = 117
GRID = 50
IMG = 800.0
HUMAN_IDX = 1
NMS_THRESH = 0.5
L = 16
NSLC = KPAD // L


def _sread(ref, i):
    return ref[pl.ds(i, L)][0]


def _swrite(ref, i, val):
    idx = jnp.full((L,), i, jnp.int32)
    lane = lax.broadcasted_iota(jnp.int32, (L,), 0)
    plsc.store_scatter(ref, [idx], jnp.full((L,), val), mask=lane == 0)


def _sgather(ref, idx):
    return plsc.load_gather(ref, [jnp.full((L,), idx, jnp.int32)])[0]


def _sc_nms_gather(boxes_p, sc_u, lb_u, cell_u, order_p, feat):
    mesh = plsc.VectorSubcoreMesh(
        core_axis_name="c", subcore_axis_name="s", num_cores=2, num_subcores=16
    )

    @functools.partial(
        pl.kernel,
        out_type=[
            jax.ShapeDtypeStruct((KPAD, D), jnp.float32),
            jax.ShapeDtypeStruct((KPAD,), jnp.float32),
            jax.ShapeDtypeStruct((KPAD,), jnp.float32),
        ],
        mesh=mesh,
        scratch_types=[
            pltpu.VMEM((NPAD * 4,), jnp.float32),
            pltpu.VMEM((NPAD,), jnp.float32),
            pltpu.VMEM((NPAD,), jnp.int32),
            pltpu.VMEM((NPAD,), jnp.int32),
            pltpu.VMEM((NPAD,), jnp.int32),
            pltpu.VMEM((KPAD,), jnp.float32),
            pltpu.VMEM((KPAD,), jnp.float32),
            pltpu.VMEM((KPAD,), jnp.float32),
            pltpu.VMEM((KPAD,), jnp.float32),
            pltpu.VMEM((KPAD,), jnp.float32),
            pltpu.VMEM((KPAD,), jnp.float32),
            pltpu.VMEM((KPAD,), jnp.float32),
            pltpu.VMEM((KPAD,), jnp.int32),
            pltpu.VMEM((KPAD, D), jnp.float32),
            pltpu.SemaphoreType.DMA,
        ],
        compiler_params=pltpu.CompilerParams(needs_layout_passes=False),
    )
    def k(h_bx, h_sc, h_lb, h_cell, h_order, h_feat,
          o_f, o_ph, o_po,
          v_bx, v_sc, v_lb, v_cell_all, v_order,
          g_x1, g_y1, g_x2, g_y2, g_a, v_ph, v_po, v_cell, v_frows, sem):
        wid = lax.axis_index("s") * 2 + lax.axis_index("c")

        @pl.when(wid == 0)
        def _():
            cps = [
                pltpu.async_copy(h_bx, v_bx, sem),
                pltpu.async_copy(h_sc, v_sc, sem),
                pltpu.async_copy(h_lb, v_lb, sem),
                pltpu.async_copy(h_cell, v_cell_all, sem),
                pltpu.async_copy(h_order, v_order, sem),
            ]
            zf = jnp.zeros((L,), jnp.float32)
            zi = jnp.zeros((L,), jnp.int32)
            for g in range(NSLC):
                sl = pl.ds(g * L, L)
                g_x1[sl] = zf
                g_y1[sl] = zf
                g_x2[sl] = zf
                g_y2[sl] = zf
                g_a[sl] = zf
                v_ph[sl] = zf
                v_po[sl] = zf
                v_cell[sl] = zi
            for c in cps:
                c.wait()

            col = jnp.bitwise_and(lax.broadcasted_iota(jnp.int32, (L,), 0), 3)

            def cond(st):
                i, cnt = st
                return jnp.logical_and(i < N, cnt < K)

            def body(st):
                i, cnt = st
                idx = _sread(v_order, i)
                c4 = plsc.load_gather(
                    v_bx, [jnp.full((L,), idx * 4, jnp.int32) + col])
                lb = _sgather(v_lb, idx)
                off = lb.astype(jnp.float32) * (IMG + 2.0)
                x1 = c4[0] + off
                y1 = c4[1] + off
                x2 = c4[2] + off
                y2 = c4[3] + off
                a = (x2 - x1) * (y2 - y1)

                nsw = (cnt + (L - 1)) >> 4

                def sweep(g, m):
                    sl = pl.ds(pl.multiple_of(g * L, L), L)
                    ix = jnp.maximum(
                        jnp.minimum(x2, g_x2[sl]) - jnp.maximum(x1, g_x1[sl]), 0.0)
                    iy = jnp.maximum(
                        jnp.minimum(y2, g_y2[sl]) - jnp.maximum(y1, g_y1[sl]), 0.0)
                    inter = ix * iy
                    denom = a + g_a[sl] - inter + 1e-8
                    return jnp.maximum(m, inter - NMS_THRESH * denom)

                m = lax.fori_loop(0, nsw, sweep, jnp.zeros((L,), jnp.float32))
                supp = jnp.max(m, axis=0) > 0.0

                @pl.when(jnp.logical_not(supp))
                def _acc():
                    _swrite(g_x1, cnt, x1)
                    _swrite(g_y1, cnt, y1)
                    _swrite(g_x2, cnt, x2)
                    _swrite(g_y2, cnt, y2)
                    _swrite(g_a, cnt, a)
                    s = _sgather(v_sc, idx)
                    _swrite(v_ph, cnt, jnp.where(lb == HUMAN_IDX, s, 0.0))
                    _swrite(v_po, cnt, s)
                    _swrite(v_cell, cnt, _sgather(v_cell_all, idx))

                return (i + 1, jnp.where(supp, cnt, cnt + 1))

            lax.while_loop(cond, body, (jnp.int32(0), jnp.int32(0)))

            pltpu.async_copy(h_feat.at[v_cell], v_frows, sem).wait()
            pltpu.sync_copy(v_frows, o_f)
            pltpu.sync_copy(v_ph, o_ph)
            pltpu.sync_copy(v_po, o_po)

    return k(boxes_p, sc_u, lb_u, cell_u, order_p, feat)


def _tc_pair(f, Wp, ph_col, po_row):
    def body(f_ref, wp_ref, ph_ref, po_ref, out_ref):
        fv = f_ref[...]
        A = jnp.dot(fv, wp_ref[0:D, :], preferred_element_type=jnp.float32)
        B = jnp.dot(fv, wp_ref[D:2 * D, :], preferred_element_type=jnp.float32)
        logits = A[0:K][:, None, :] + B[0:K][None, :, :]
        prior = ph_ref[0:K, :] * po_ref[:, 0:K]
        out_ref[...] = (1.0 / (1.0 + jnp.exp(-logits))) * prior[:, :, None]

    return pl.pallas_call(
        body,
        out_shape=jax.ShapeDtypeStruct((K, K, NI), jnp.float32),
    )(f, Wp, ph_col, po_row)


def kernel(boxes, scores, labels, feat_map, Wp):
    order = jnp.argsort(-lax.stop_gradient(scores)).astype(jnp.int32)
    cx = lax.stop_gradient((boxes[:, 0] + boxes[:, 2]) * 0.5)
    cy = lax.stop_gradient((boxes[:, 1] + boxes[:, 3]) * 0.5)
    gx = jnp.clip((cx / IMG * GRID).astype(jnp.int32), 0, GRID - 1)
    gy = jnp.clip((cy / IMG * GRID).astype(jnp.int32), 0, GRID - 1)
    cell = gy * GRID + gx

    boxes_p = jnp.pad(boxes, ((0, NPAD - N), (0, 0))).reshape(-1)
    f, ph, po = _sc_nms_gather(
        boxes_p,
        jnp.pad(scores, (0, NPAD - N)),
        jnp.pad(labels, (0, NPAD - N)),
        jnp.pad(cell, (0, NPAD - N)),
        jnp.pad(order, (0, NPAD - N)),
        feat_map)
    out3 = _tc_pair(f, Wp, ph.reshape(KPAD, 1), po.reshape(1, KPAD))
    return out3.reshape(K * K, NI)
